# Initial kernel scaffold; baseline (speedup 1.0000x reference)
import jax
import jax.numpy as jnp
from jax.experimental import pallas as pl


def kernel(query_embedding, semantic, structural, contextual, W, b, top_k):
    raise NotImplementedError("write your pallas kernel here")


# speedup vs baseline: 3.1775x; 3.1775x over previous
import functools

import jax
import jax.numpy as jnp
from jax.experimental import pallas as pl
from jax.experimental.pallas import tpu as pltpu

HD = 128
N_BIG = 100000
N_STR = 10000
BLK = 2048
NS_GRID = 49
SROWS = 56
STR_BLKS = 5
K = 32
NEG_INF = float("-inf")


def _router_kernel(query_ref, w_ref, b_ref, sem_ref, ctx_ref, str_ref,
                   sem_vals_ref, sem_idx_ref, ctx_vals_ref, ctx_idx_ref,
                   str_sims_ref, q_scr, sem_scr, ctx_scr):
    i = pl.program_id(0)

    @pl.when(i == 0)
    def _init():
        q = jax.lax.dot_general(
            query_ref[...], w_ref[...], (((1,), (1,)), ((), ())),
            preferred_element_type=jnp.float32)
        q_scr[...] = q + b_ref[...]
        sem_scr[...] = jnp.full((SROWS, BLK), NEG_INF, jnp.float32)
        ctx_scr[...] = jnp.full((SROWS, BLK), NEG_INF, jnp.float32)

    q = q_scr[...]

    lane = jax.lax.broadcasted_iota(jnp.int32, (1, BLK), 1)
    valid = (i * BLK + lane) < N_BIG

    sem_sim = jax.lax.dot_general(
        q, sem_ref[...], (((1,), (1,)), ((), ())),
        preferred_element_type=jnp.float32)
    sem_scr[pl.ds(i, 1), :] = jnp.where(valid, sem_sim, NEG_INF)

    ctx_sim = jax.lax.dot_general(
        q, ctx_ref[...], (((1,), (1,)), ((), ())),
        preferred_element_type=jnp.float32)
    ctx_scr[pl.ds(i, 1), :] = jnp.where(valid, ctx_sim, NEG_INF)

    @pl.when(i < STR_BLKS)
    def _str():
        s = jax.lax.dot_general(
            q, str_ref[...], (((1,), (1,)), ((), ())),
            preferred_element_type=jnp.float32)
        str_sims_ref[...] = s

    @pl.when(i == NS_GRID - 1)
    def _extract():
        idx2 = (jax.lax.broadcasted_iota(jnp.int32, (SROWS, BLK), 0) * BLK
                + jax.lax.broadcasted_iota(jnp.int32, (SROWS, BLK), 1))
        kidx = jax.lax.broadcasted_iota(jnp.int32, (1, K), 1)

        def topk(scr, vref, iref):
            def body(j, carry):
                s, vals, inds = carry
                m = jnp.max(s)
                am = jnp.min(jnp.where(s == m, idx2, jnp.int32(2**31 - 1)))
                s = jnp.where(idx2 == am, NEG_INF, s)
                vals = jnp.where(kidx == j, m, vals)
                inds = jnp.where(kidx == j, am, inds)
                return s, vals, inds

            _, vals, inds = jax.lax.fori_loop(
                0, K, body,
                (scr[...], jnp.zeros((1, K), jnp.float32),
                 jnp.zeros((1, K), jnp.int32)))
            vref[...] = vals
            iref[...] = inds

        topk(sem_scr, sem_vals_ref, sem_idx_ref)
        topk(ctx_scr, ctx_vals_ref, ctx_idx_ref)


def _gather_kernel(idx_ref, sem_ref, ctx_ref, out_ref):
    i = pl.program_id(0)
    out_ref[...] = jnp.where(i < K, sem_ref[...], ctx_ref[...])


@jax.jit
def _run(query_embedding, semantic, structural, contextual, W, b):
    b2 = b.reshape(1, HD)
    out_shapes = [
        jax.ShapeDtypeStruct((1, K), jnp.float32),
        jax.ShapeDtypeStruct((1, K), jnp.int32),
        jax.ShapeDtypeStruct((1, K), jnp.float32),
        jax.ShapeDtypeStruct((1, K), jnp.int32),
        jax.ShapeDtypeStruct((1, STR_BLKS * BLK), jnp.float32),
    ]
    in_specs = [
        pl.BlockSpec((1, HD), lambda i: (0, 0)),
        pl.BlockSpec((HD, HD), lambda i: (0, 0)),
        pl.BlockSpec((1, HD), lambda i: (0, 0)),
        pl.BlockSpec((BLK, HD), lambda i: (i, 0)),
        pl.BlockSpec((BLK, HD), lambda i: (i, 0)),
        pl.BlockSpec((BLK, HD), lambda i: (jnp.minimum(i, STR_BLKS - 1), 0)),
    ]
    out_specs = [
        pl.BlockSpec((1, K), lambda i: (0, 0)),
        pl.BlockSpec((1, K), lambda i: (0, 0)),
        pl.BlockSpec((1, K), lambda i: (0, 0)),
        pl.BlockSpec((1, K), lambda i: (0, 0)),
        pl.BlockSpec((1, BLK), lambda i: (0, jnp.minimum(i, STR_BLKS - 1))),
    ]
    scratch = [
        pltpu.VMEM((1, HD), jnp.float32),
        pltpu.VMEM((SROWS, BLK), jnp.float32),
        pltpu.VMEM((SROWS, BLK), jnp.float32),
    ]
    sem_vals, sem_idx, ctx_vals, ctx_idx, str_sims = pl.pallas_call(
        _router_kernel,
        grid=(NS_GRID,),
        in_specs=in_specs,
        out_specs=out_specs,
        out_shape=out_shapes,
        scratch_shapes=scratch,
    )(query_embedding, W, b2, semantic, contextual, structural)

    all_idx = jnp.concatenate([sem_idx[0], ctx_idx[0]])
    grid_spec = pltpu.PrefetchScalarGridSpec(
        num_scalar_prefetch=1,
        grid=(2 * K,),
        in_specs=[
            pl.BlockSpec((1, 1, HD), lambda i, idx: (idx[i], 0, 0)),
            pl.BlockSpec((1, 1, HD), lambda i, idx: (idx[i], 0, 0)),
        ],
        out_specs=pl.BlockSpec((1, 1, HD), lambda i, idx: (i, 0, 0)),
    )
    sel = pl.pallas_call(
        _gather_kernel,
        grid_spec=grid_spec,
        out_shape=jax.ShapeDtypeStruct((2 * K, 1, HD), jnp.float32),
    )(all_idx, semantic.reshape(-1, 1, HD), contextual.reshape(-1, 1, HD))
    sel = sel.reshape(2 * K, HD)

    all_weights = jnp.concatenate(
        [sem_vals[0], str_sims[0, :N_STR], ctx_vals[0]])
    return sel[:K], structural, sel[K:], all_weights


def kernel(query_embedding, semantic, structural, contextual, W, b, top_k):
    return _run(query_embedding, semantic, structural, contextual, W, b)
